# baseline (device time: 26550 ns/iter reference)
import jax
import jax.numpy as jnp
from jax import lax
from jax.experimental import pallas as pl
from jax.experimental.pallas import tpu as pltpu

M = 2048
N = 1024
HALF = 512
C = 16
CH = HALF // C
EPS = 1e-6


def kernel(partial, gamma):
    p = partial.reshape(M, N)
    my_x = lax.axis_index("x")
    my_y = lax.axis_index("y")
    send_base = (1 - my_y) * (M // 2) + my_x * HALF
    loc_base = my_y * (M // 2) + my_x * HALF
    p_send = lax.dynamic_slice(p, (send_base, 0), (HALF, N)).astype(jnp.bfloat16)
    p_loc = lax.dynamic_slice(p, (loc_base, 0), (HALF, N)).astype(jnp.bfloat16)
    g = gamma.reshape(1, N)

    def body(ps_ref, pl_ref, g_ref, out_ref, recv_y, fwd_x, recv_x,
             sem_sy, sem_ry, sem_sx, sem_rx):
        my_x = lax.axis_index("x")
        my_y = lax.axis_index("y")
        y_nbr = (my_x, 1 - my_y)
        x_nbr = (1 - my_x, my_y)

        barrier = pltpu.get_barrier_semaphore()
        for nbr in (y_nbr, x_nbr):
            pl.semaphore_signal(
                barrier, inc=1, device_id=nbr,
                device_id_type=pl.DeviceIdType.MESH,
            )
        pl.semaphore_wait(barrier, 2)

        y_rdmas = []
        for c in range(C):
            sl = pl.ds(c * CH, CH)
            r = pltpu.make_async_remote_copy(
                src_ref=ps_ref.at[sl, :],
                dst_ref=recv_y.at[sl, :],
                send_sem=sem_sy.at[c],
                recv_sem=sem_ry.at[c],
                device_id=y_nbr,
                device_id_type=pl.DeviceIdType.MESH,
            )
            r.start()
            y_rdmas.append(r)

        def norm_x_chunk(c):
            sl = pl.ds(c * CH, CH)
            x_rdmas[c].wait_recv()
            f = recv_x[sl, :].astype(jnp.float32)
            scale = lax.rsqrt(jnp.mean(f * f, axis=-1, keepdims=True) + EPS)
            out_ref[pl.ds((1 - my_x) * HALF + c * CH, CH), :] = (
                f * (scale * g_ref[...])
            )

        LAG = 2
        x_rdmas = []
        for c in range(C):
            sl = pl.ds(c * CH, CH)
            y_rdmas[c].wait_recv()
            s = recv_y[sl, :] + pl_ref[sl, :]
            fwd_x[sl, :] = s
            r = pltpu.make_async_remote_copy(
                src_ref=fwd_x.at[sl, :],
                dst_ref=recv_x.at[sl, :],
                send_sem=sem_sx.at[c],
                recv_sem=sem_rx.at[c],
                device_id=x_nbr,
                device_id_type=pl.DeviceIdType.MESH,
            )
            r.start()
            x_rdmas.append(r)
            f = s.astype(jnp.float32)
            scale = lax.rsqrt(jnp.mean(f * f, axis=-1, keepdims=True) + EPS)
            out_ref[pl.ds(my_x * HALF + c * CH, CH), :] = f * (scale * g_ref[...])
            if c >= LAG:
                norm_x_chunk(c - LAG)

        for c in range(C - LAG, C):
            norm_x_chunk(c)

        for c in range(C):
            y_rdmas[c].wait_send()
            x_rdmas[c].wait_send()

    return pl.pallas_call(
        body,
        out_shape=jax.ShapeDtypeStruct((M // 2, N), jnp.float32),
        in_specs=[
            pl.BlockSpec(memory_space=pltpu.VMEM),
            pl.BlockSpec(memory_space=pltpu.VMEM),
            pl.BlockSpec(memory_space=pltpu.VMEM),
        ],
        out_specs=pl.BlockSpec(memory_space=pltpu.VMEM),
        scratch_shapes=[
            pltpu.VMEM((HALF, N), jnp.bfloat16),
            pltpu.VMEM((HALF, N), jnp.bfloat16),
            pltpu.VMEM((HALF, N), jnp.bfloat16),
            pltpu.SemaphoreType.DMA((C,)),
            pltpu.SemaphoreType.DMA((C,)),
            pltpu.SemaphoreType.DMA((C,)),
            pltpu.SemaphoreType.DMA((C,)),
        ],
        compiler_params=pltpu.CompilerParams(collective_id=0),
    )(p_send, p_loc, g)


# device time: 25942 ns/iter; 1.0234x vs baseline; 1.0234x over previous
import jax
import jax.numpy as jnp
from jax import lax
from jax.experimental import pallas as pl
from jax.experimental.pallas import tpu as pltpu

M = 2048
N = 1024
HALF = 512
C = 4
CH = HALF // C
EPS = 1e-6


def kernel(partial, gamma):
    p = partial.reshape(M, N)
    my_x = lax.axis_index("x")
    my_y = lax.axis_index("y")
    send_base = (1 - my_y) * (M // 2) + my_x * HALF
    loc_base = my_y * (M // 2) + my_x * HALF
    p_send = lax.dynamic_slice(p, (send_base, 0), (HALF, N)).astype(jnp.bfloat16)
    p_loc = lax.dynamic_slice(p, (loc_base, 0), (HALF, N)).astype(jnp.bfloat16)
    g = gamma.reshape(1, N)

    def body(ps_ref, pl_ref, g_ref, out_ref, recv_y, fwd_x, recv_x,
             sem_sy, sem_ry, sem_sx, sem_rx):
        my_x = lax.axis_index("x")
        my_y = lax.axis_index("y")
        y_nbr = (my_x, 1 - my_y)
        x_nbr = (1 - my_x, my_y)

        barrier = pltpu.get_barrier_semaphore()
        for nbr in (y_nbr, x_nbr):
            pl.semaphore_signal(
                barrier, inc=1, device_id=nbr,
                device_id_type=pl.DeviceIdType.MESH,
            )
        pl.semaphore_wait(barrier, 2)

        y_rdmas = []
        for c in range(C):
            sl = pl.ds(c * CH, CH)
            r = pltpu.make_async_remote_copy(
                src_ref=ps_ref.at[sl, :],
                dst_ref=recv_y.at[sl, :],
                send_sem=sem_sy.at[c],
                recv_sem=sem_ry.at[c],
                device_id=y_nbr,
                device_id_type=pl.DeviceIdType.MESH,
            )
            r.start()
            y_rdmas.append(r)

        def norm_x_chunk(c):
            sl = pl.ds(c * CH, CH)
            x_rdmas[c].wait_recv()
            f = recv_x[sl, :].astype(jnp.float32)
            scale = lax.rsqrt(jnp.mean(f * f, axis=-1, keepdims=True) + EPS)
            out_ref[pl.ds((1 - my_x) * HALF + c * CH, CH), :] = (
                f * (scale * g_ref[...])
            )

        LAG = 2
        x_rdmas = []
        for c in range(C):
            sl = pl.ds(c * CH, CH)
            y_rdmas[c].wait_recv()
            s = recv_y[sl, :] + pl_ref[sl, :]
            fwd_x[sl, :] = s
            r = pltpu.make_async_remote_copy(
                src_ref=fwd_x.at[sl, :],
                dst_ref=recv_x.at[sl, :],
                send_sem=sem_sx.at[c],
                recv_sem=sem_rx.at[c],
                device_id=x_nbr,
                device_id_type=pl.DeviceIdType.MESH,
            )
            r.start()
            x_rdmas.append(r)
            f = s.astype(jnp.float32)
            scale = lax.rsqrt(jnp.mean(f * f, axis=-1, keepdims=True) + EPS)
            out_ref[pl.ds(my_x * HALF + c * CH, CH), :] = f * (scale * g_ref[...])
            if c >= LAG:
                norm_x_chunk(c - LAG)

        for c in range(C - LAG, C):
            norm_x_chunk(c)

        for c in range(C):
            y_rdmas[c].wait_send()
            x_rdmas[c].wait_send()

    return pl.pallas_call(
        body,
        out_shape=jax.ShapeDtypeStruct((M // 2, N), jnp.float32),
        in_specs=[
            pl.BlockSpec(memory_space=pltpu.VMEM),
            pl.BlockSpec(memory_space=pltpu.VMEM),
            pl.BlockSpec(memory_space=pltpu.VMEM),
        ],
        out_specs=pl.BlockSpec(memory_space=pltpu.VMEM),
        scratch_shapes=[
            pltpu.VMEM((HALF, N), jnp.bfloat16),
            pltpu.VMEM((HALF, N), jnp.bfloat16),
            pltpu.VMEM((HALF, N), jnp.bfloat16),
            pltpu.SemaphoreType.DMA((C,)),
            pltpu.SemaphoreType.DMA((C,)),
            pltpu.SemaphoreType.DMA((C,)),
            pltpu.SemaphoreType.DMA((C,)),
        ],
        compiler_params=pltpu.CompilerParams(collective_id=0),
    )(p_send, p_loc, g)


# device time: 23999 ns/iter; 1.1063x vs baseline; 1.0810x over previous
import jax
import jax.numpy as jnp
from jax import lax
from jax.experimental import pallas as pl
from jax.experimental.pallas import tpu as pltpu

M = 2048
N = 1024
HALF = 512
C = 8
CH = HALF // C
EPS = 1e-6


def kernel(partial, gamma):
    p = partial.reshape(M, N)
    my_x = lax.axis_index("x")
    my_y = lax.axis_index("y")
    send_base = (1 - my_y) * (M // 2) + my_x * HALF
    loc_base = my_y * (M // 2) + my_x * HALF
    p_send = lax.dynamic_slice(p, (send_base, 0), (HALF, N)).astype(jnp.bfloat16)
    p_loc = lax.dynamic_slice(p, (loc_base, 0), (HALF, N)).astype(jnp.bfloat16)
    g = gamma.reshape(1, N)

    def body(ps_ref, pl_ref, g_ref, out_ref, recv_y, fwd_x, recv_x,
             sem_sy, sem_ry, sem_sx, sem_rx):
        my_x = lax.axis_index("x")
        my_y = lax.axis_index("y")
        y_nbr = (my_x, 1 - my_y)
        x_nbr = (1 - my_x, my_y)

        barrier = pltpu.get_barrier_semaphore()
        for nbr in (y_nbr, x_nbr):
            pl.semaphore_signal(
                barrier, inc=1, device_id=nbr,
                device_id_type=pl.DeviceIdType.MESH,
            )
        pl.semaphore_wait(barrier, 2)

        y_rdmas = []
        for c in range(C):
            sl = pl.ds(c * CH, CH)
            r = pltpu.make_async_remote_copy(
                src_ref=ps_ref.at[sl, :],
                dst_ref=recv_y.at[sl, :],
                send_sem=sem_sy.at[c],
                recv_sem=sem_ry.at[c],
                device_id=y_nbr,
                device_id_type=pl.DeviceIdType.MESH,
            )
            r.start()
            y_rdmas.append(r)

        def norm_x_chunk(c):
            sl = pl.ds(c * CH, CH)
            x_rdmas[c].wait_recv()
            f = recv_x[sl, :].astype(jnp.float32)
            scale = lax.rsqrt(jnp.mean(f * f, axis=-1, keepdims=True) + EPS)
            out_ref[pl.ds((1 - my_x) * HALF + c * CH, CH), :] = (
                f * (scale * g_ref[...])
            ).astype(jnp.bfloat16)

        LAG = 2
        x_rdmas = []
        for c in range(C):
            sl = pl.ds(c * CH, CH)
            y_rdmas[c].wait_recv()
            s = recv_y[sl, :] + pl_ref[sl, :]
            fwd_x[sl, :] = s
            r = pltpu.make_async_remote_copy(
                src_ref=fwd_x.at[sl, :],
                dst_ref=recv_x.at[sl, :],
                send_sem=sem_sx.at[c],
                recv_sem=sem_rx.at[c],
                device_id=x_nbr,
                device_id_type=pl.DeviceIdType.MESH,
            )
            r.start()
            x_rdmas.append(r)
            f = s.astype(jnp.float32)
            scale = lax.rsqrt(jnp.mean(f * f, axis=-1, keepdims=True) + EPS)
            out_ref[pl.ds(my_x * HALF + c * CH, CH), :] = (
                f * (scale * g_ref[...])
            ).astype(jnp.bfloat16)
            if c >= LAG:
                norm_x_chunk(c - LAG)

        for c in range(C - LAG, C):
            norm_x_chunk(c)

        for c in range(C):
            y_rdmas[c].wait_send()
            x_rdmas[c].wait_send()

    return pl.pallas_call(
        body,
        out_shape=jax.ShapeDtypeStruct((M // 2, N), jnp.bfloat16),
        in_specs=[
            pl.BlockSpec(memory_space=pltpu.VMEM),
            pl.BlockSpec(memory_space=pltpu.VMEM),
            pl.BlockSpec(memory_space=pltpu.VMEM),
        ],
        out_specs=pl.BlockSpec(memory_space=pltpu.VMEM),
        scratch_shapes=[
            pltpu.VMEM((HALF, N), jnp.bfloat16),
            pltpu.VMEM((HALF, N), jnp.bfloat16),
            pltpu.VMEM((HALF, N), jnp.bfloat16),
            pltpu.SemaphoreType.DMA((C,)),
            pltpu.SemaphoreType.DMA((C,)),
            pltpu.SemaphoreType.DMA((C,)),
            pltpu.SemaphoreType.DMA((C,)),
        ],
        compiler_params=pltpu.CompilerParams(collective_id=0),
    )(p_send, p_loc, g)


# device time: 23988 ns/iter; 1.1068x vs baseline; 1.0005x over previous
import jax
import jax.numpy as jnp
from jax import lax
from jax.experimental import pallas as pl
from jax.experimental.pallas import tpu as pltpu

M = 2048
N = 1024
HALF = 512
C = 8
CH = HALF // C
EPS = 1e-6


def kernel(partial, gamma):
    my_x = lax.axis_index("x")
    my_y = lax.axis_index("y")
    send_base = (1 - my_y) * (M // 2) + my_x * HALF
    loc_base = my_y * (M // 2) + my_x * HALF
    p_send = lax.dynamic_slice(
        partial, (0, send_base, 0), (1, HALF, N)
    ).astype(jnp.bfloat16)
    p_loc = lax.dynamic_slice(
        partial, (0, loc_base, 0), (1, HALF, N)
    ).astype(jnp.bfloat16)
    g = gamma.reshape(1, N)

    def body(ps_ref, pl_ref, g_ref, out_ref, recv_y, fwd_x, recv_x,
             sem_sy, sem_ry, sem_sx, sem_rx):
        my_x = lax.axis_index("x")
        my_y = lax.axis_index("y")
        y_nbr = (my_x, 1 - my_y)
        x_nbr = (1 - my_x, my_y)

        barrier = pltpu.get_barrier_semaphore()
        for nbr in (y_nbr, x_nbr):
            pl.semaphore_signal(
                barrier, inc=1, device_id=nbr,
                device_id_type=pl.DeviceIdType.MESH,
            )
        pl.semaphore_wait(barrier, 2)

        y_rdmas = []
        for c in range(C):
            sl = pl.ds(c * CH, CH)
            r = pltpu.make_async_remote_copy(
                src_ref=ps_ref.at[0, sl, :],
                dst_ref=recv_y.at[sl, :],
                send_sem=sem_sy.at[c],
                recv_sem=sem_ry.at[c],
                device_id=y_nbr,
                device_id_type=pl.DeviceIdType.MESH,
            )
            r.start()
            y_rdmas.append(r)

        def norm_x_chunk(c):
            sl = pl.ds(c * CH, CH)
            x_rdmas[c].wait_recv()
            f = recv_x[sl, :].astype(jnp.float32)
            scale = lax.rsqrt(jnp.mean(f * f, axis=-1, keepdims=True) + EPS)
            out_ref[pl.ds((1 - my_x) * HALF + c * CH, CH), :] = (
                f * (scale * g_ref[...])
            ).astype(jnp.bfloat16)

        LAG = 2
        x_rdmas = []
        for c in range(C):
            sl = pl.ds(c * CH, CH)
            y_rdmas[c].wait_recv()
            s = recv_y[sl, :] + pl_ref[0, sl, :]
            fwd_x[sl, :] = s
            r = pltpu.make_async_remote_copy(
                src_ref=fwd_x.at[sl, :],
                dst_ref=recv_x.at[sl, :],
                send_sem=sem_sx.at[c],
                recv_sem=sem_rx.at[c],
                device_id=x_nbr,
                device_id_type=pl.DeviceIdType.MESH,
            )
            r.start()
            x_rdmas.append(r)
            f = s.astype(jnp.float32)
            scale = lax.rsqrt(jnp.mean(f * f, axis=-1, keepdims=True) + EPS)
            out_ref[pl.ds(my_x * HALF + c * CH, CH), :] = (
                f * (scale * g_ref[...])
            ).astype(jnp.bfloat16)
            if c >= LAG:
                norm_x_chunk(c - LAG)

        for c in range(C - LAG, C):
            norm_x_chunk(c)

        for c in range(C):
            y_rdmas[c].wait_send()
            x_rdmas[c].wait_send()

    return pl.pallas_call(
        body,
        out_shape=jax.ShapeDtypeStruct((M // 2, N), jnp.bfloat16),
        in_specs=[
            pl.BlockSpec(memory_space=pltpu.VMEM),
            pl.BlockSpec(memory_space=pltpu.VMEM),
            pl.BlockSpec(memory_space=pltpu.VMEM),
        ],
        out_specs=pl.BlockSpec(memory_space=pltpu.VMEM),
        scratch_shapes=[
            pltpu.VMEM((HALF, N), jnp.bfloat16),
            pltpu.VMEM((HALF, N), jnp.bfloat16),
            pltpu.VMEM((HALF, N), jnp.bfloat16),
            pltpu.SemaphoreType.DMA((C,)),
            pltpu.SemaphoreType.DMA((C,)),
            pltpu.SemaphoreType.DMA((C,)),
            pltpu.SemaphoreType.DMA((C,)),
        ],
        compiler_params=pltpu.CompilerParams(collective_id=0),
    )(p_send, p_loc, g)
